# baseline (speedup 1.0000x reference)
import functools

import jax
import jax.numpy as jnp
from jax import lax
from jax.experimental import pallas as pl
from jax.experimental.pallas import tpu as pltpu
from jax.experimental.pallas import tpu_sc as plsc

_N = 10000
_E = 160000
_D = 256
_DE = 16
_HID = 512

_NC = 2
_NS = 16
_NW = _NC * _NS

_sc_params = pltpu.CompilerParams(needs_layout_passes=False)
_sc_mesh = plsc.VectorSubcoreMesh(core_axis_name="c", subcore_axis_name="s")

_GB = 2 * _E
_GPW = _GB // _NW
_GK = 80
_GCH = _GPW // _GK


@functools.partial(
    pl.kernel,
    mesh=_sc_mesh,
    compiler_params=_sc_params,
    out_type=jax.ShapeDtypeStruct((_GB, _D), jnp.float32),
    scratch_types=[
        pltpu.VMEM((_GPW,), jnp.int32),
        pltpu.VMEM((_GK, _D), jnp.float32),
        pltpu.VMEM((_GK, _D), jnp.float32),
        pltpu.SemaphoreType.DMA,
        pltpu.SemaphoreType.DMA,
        pltpu.SemaphoreType.DMA,
        pltpu.SemaphoreType.DMA,
    ],
)
def _sc_gather(x_hbm, idx_hbm, out_hbm, idx_v, rowsA, rowsB,
               gsemA, gsemB, wsemA, wsemB):
    c = lax.axis_index("c")
    s = lax.axis_index("s")
    wid = s * _NC + c
    base = wid * _GPW
    pltpu.sync_copy(idx_hbm.at[pl.ds(base, _GPW)], idx_v)
    pltpu.async_copy(x_hbm.at[idx_v.at[pl.ds(0, _GK)]], rowsA, gsemA)

    def super_step(k2, carry):
        for b in range(2):
            chunk = k2 * 2 + b
            rowsP, gsemP, wsemP = (
                (rowsA, gsemA, wsemA) if b == 0 else (rowsB, gsemB, wsemB)
            )
            rowsQ, gsemQ, wsemQ = (
                (rowsB, gsemB, wsemB) if b == 0 else (rowsA, gsemA, wsemA)
            )
            off = chunk * _GK

            @pl.when(chunk < _GCH)
            def _():
                pltpu.make_async_copy(
                    x_hbm.at[idx_v.at[pl.ds(off, _GK)]], rowsP, gsemP
                ).wait()

                @pl.when(chunk >= 1)
                def _():
                    pltpu.make_async_copy(
                        rowsQ, out_hbm.at[pl.ds(base + off - _GK, _GK)], wsemQ
                    ).wait()

                @pl.when(chunk + 1 < _GCH)
                def _():
                    pltpu.async_copy(
                        x_hbm.at[idx_v.at[pl.ds(off + _GK, _GK)]],
                        rowsQ, gsemQ,
                    )

                pltpu.async_copy(
                    rowsP, out_hbm.at[pl.ds(base + off, _GK)], wsemP
                )

            @pl.when(chunk == _GCH)
            def _():
                pltpu.make_async_copy(
                    rowsQ, out_hbm.at[pl.ds(base + off - _GK, _GK)], wsemQ
                ).wait()

        return carry

    lax.fori_loop(0, _GCH // 2 + 1, super_step, 0)


_OWN = 320
_SCC = 1600
_NCH = _E // _SCC
_SK = 48
_EC = _SCC + _SK
_AR = 336


@functools.partial(
    pl.kernel,
    mesh=_sc_mesh,
    compiler_params=_sc_params,
    out_type=(
        jax.ShapeDtypeStruct((_NW * _OWN, _D), jnp.float32),
        jax.ShapeDtypeStruct((_NW * _OWN,), jnp.float32),
    ),
    scratch_types=[
        pltpu.VMEM((_SCC,), jnp.int32),
        pltpu.VMEM((_SCC,), jnp.int32),
        pltpu.VMEM((_EC,), jnp.int32),
        pltpu.VMEM((_EC,), jnp.int32),
        pltpu.VMEM((_EC,), jnp.int32),
        pltpu.VMEM((_EC,), jnp.int32),
        pltpu.VMEM((_SK, _D), jnp.float32),
        pltpu.VMEM((_SK, _D), jnp.float32),
        pltpu.VMEM((_AR, _D), jnp.float32),
        pltpu.VMEM((16 * _OWN,), jnp.float32),
        pltpu.VMEM((_OWN,), jnp.float32),
        pltpu.SemaphoreType.DMA,
        pltpu.SemaphoreType.DMA,
        pltpu.SemaphoreType.DMA,
        pltpu.SemaphoreType.DMA,
    ],
)
def _sc_scatter(msg_hbm, dst_hbm, acc_out, cnt_out,
                dstA, dstB, elA, elB, llA, llB, mbA, mbB,
                acc, lcnt, cntf, dsemA, dsemB, msemA, msemB):
    c = lax.axis_index("c")
    s = lax.axis_index("s")
    wid = s * _NC + c
    nbase = wid * _OWN
    zero16 = jnp.zeros((16,), jnp.float32)
    ones16 = jnp.ones((16,), jnp.float32)
    iota16 = lax.iota(jnp.int32, 16)

    def zrow(r, carry):
        for l in range(_D // 16):
            acc[r, pl.ds(l * 16, 16)] = zero16
        return carry

    lax.fori_loop(0, _AR, zrow, 0)

    def zcnt(r, carry):
        lcnt[pl.ds(r * 16, 16)] = zero16
        return carry

    lax.fori_loop(0, 16 * _OWN // 16, zcnt, 0)

    def compact(chunk, dstb, elb, llb, msem, mbb):
        ebase = chunk * _SCC

        def grp(g, carry):
            pos, evec = carry
            for u in range(2):
                dv = dstb[pl.ds(g * 32 + u * 16, 16)]
                ok = (dv >= nbase) & (dv < nbase + _OWN)
                lid = jnp.where(ok, dv - nbase, 0)
                plsc.addupdate_scatter(lcnt, [iota16 * _OWN + lid], ones16,
                                       mask=ok)
                rank = plsc.cumsum(
                    jnp.where(ok, jnp.int32(1), jnp.int32(0))) - 1
                slot = jnp.maximum(pos + rank, 0)
                plsc.store_scatter(elb, [slot], evec, mask=ok)
                pc = plsc.all_reduce_population_count(ok)
                pos = pos + pc
                evec = evec + 16
            return pos, evec

        pos, _ = lax.fori_loop(0, _SCC // 32, grp,
                               (jnp.zeros((16,), jnp.int32), ebase + iota16))
        m = jnp.max(pos)
        padv = jnp.full((16,), ebase, jnp.int32)
        for j in range(_SK // 16):
            plsc.store_scatter(elb, [m + j * 16 + iota16], padv)

        def lids(r, carry):
            eidv = elb[pl.ds(r * 16, 16)]
            lid16 = plsc.load_gather(dstb, [eidv - ebase]) - nbase
            llb[pl.ds(r * 16, 16)] = lid16
            return carry

        lax.fori_loop(0, (m + 15) // 16, lids, 0)
        dumv = jnp.full((16,), _OWN, jnp.int32)
        for j in range(_SK // 16):
            plsc.store_scatter(llb, [m + j * 16 + iota16], dumv)
        cp = pltpu.async_copy(msg_hbm.at[elb.at[pl.ds(0, _SK)]], mbb, msem)
        del cp
        return m

    def accumulate(m, elb, llb, msem, mbb):
        pltpu.make_async_copy(msg_hbm.at[elb.at[pl.ds(0, _SK)]], mbb,
                              msem).wait()

        def dorows(t, carry):
            def row(j, carry2):
                for u in range(2):
                    jj = j * 2 + u
                    rid = plsc.load_gather(
                        llb, [jnp.full((16,), t * _SK + jj, jnp.int32)]
                    )
                    for l in range(_D // 16):
                        plsc.addupdate_scatter(
                            acc, [rid, iota16 + l * 16],
                            mbb[jj, pl.ds(l * 16, 16)]
                        )
                return carry2

            lax.fori_loop(0, _SK // 2, row, 0)
            return carry

        dorows(jnp.int32(0), 0)

        def extra(t, carry):
            pltpu.async_copy(
                msg_hbm.at[elb.at[pl.ds(t * _SK, _SK)]], mbb, msem
            ).wait()
            dorows(t, carry)
            return carry

        lax.fori_loop(1, (m + _SK - 1) // _SK, extra, 0)

    pltpu.async_copy(dst_hbm.at[pl.ds(0, _SCC)], dstA, dsemA)

    def super_step(k2, m_prev):
        for b in range(2):
            chunk = k2 * 2 + b
            dstb, elb, llb, msem, mbb = (
                (dstA, elA, llA, msemA, mbA) if b == 0
                else (dstB, elB, llB, msemB, mbB)
            )
            dsem = dsemA if b == 0 else dsemB
            dsemN = dsemB if b == 0 else dsemA
            dstN = dstB if b == 0 else dstA
            elQ, llQ, msemQ, mbQ = (
                (elB, llB, msemB, mbB) if b == 0
                else (elA, llA, msemA, mbA)
            )

            @pl.when(chunk < _NCH)
            def _():
                pltpu.make_async_copy(
                    dst_hbm.at[pl.ds(chunk * _SCC, _SCC)], dstb, dsem
                ).wait()

                @pl.when(chunk + 1 < _NCH)
                def _():
                    pltpu.async_copy(
                        dst_hbm.at[pl.ds((chunk + 1) * _SCC, _SCC)],
                        dstN, dsemN,
                    )

            m_new = lax.cond(
                chunk < _NCH,
                lambda: compact(chunk, dstb, elb, llb, msem, mbb),
                lambda: jnp.int32(0),
            )

            @pl.when((chunk >= 1) & (chunk <= _NCH))
            def _():
                accumulate(m_prev, elQ, llQ, msemQ, mbQ)

            m_prev = m_new
        return m_prev

    lax.fori_loop(0, _NCH // 2 + 1, super_step, jnp.int32(0))

    def cred(g, carry):
        tot = zero16
        for lane in range(16):
            tot = tot + lcnt[pl.ds(lane * _OWN + g * 16, 16)]
        cntf[pl.ds(g * 16, 16)] = tot
        return carry

    lax.fori_loop(0, _OWN // 16, cred, 0)
    pltpu.sync_copy(acc.at[pl.ds(0, _OWN)], acc_out.at[pl.ds(nbase, _OWN)])
    pltpu.sync_copy(cntf, cnt_out.at[pl.ds(nbase, _OWN)])


_TE = 640
_GE = _E // _TE


def _edge_body(xi, xj, ea, w1a, w1b, w1c, b1, w2, b2, out):
    bf = jnp.bfloat16
    h = jnp.dot(xi[...].astype(bf), w1a[...], preferred_element_type=jnp.float32)
    h = h + jnp.dot(xj[...].astype(bf), w1b[...], preferred_element_type=jnp.float32)
    h = h + jnp.dot(ea[...], w1c[...], preferred_element_type=jnp.float32)
    h = h + b1[...]
    h = jnp.where(h > 0.0, h, jnp.exp(jnp.minimum(h, 0.0)) - 1.0)
    out[...] = jnp.dot(h.astype(bf), w2[...].astype(bf),
                       preferred_element_type=jnp.float32) + b2[...]


def _edge_mlp(gathered, edge_attr, w1a, w1b, w1c, b1, w2, b2):
    return pl.pallas_call(
        _edge_body,
        grid=(_GE,),
        in_specs=[
            pl.BlockSpec((_TE, _D), lambda i: (i, 0)),
            pl.BlockSpec((_TE, _D), lambda i: (i + _GE, 0)),
            pl.BlockSpec((_TE, _DE), lambda i: (i, 0)),
            pl.BlockSpec((_D, _HID), lambda i: (0, 0)),
            pl.BlockSpec((_D, _HID), lambda i: (0, 0)),
            pl.BlockSpec((_DE, _HID), lambda i: (0, 0)),
            pl.BlockSpec((1, _HID), lambda i: (0, 0)),
            pl.BlockSpec((_HID, _D), lambda i: (0, 0)),
            pl.BlockSpec((1, _D), lambda i: (0, 0)),
        ],
        out_specs=pl.BlockSpec((_TE, _D), lambda i: (i, 0)),
        out_shape=jax.ShapeDtypeStruct((_E, _D), jnp.float32),
        compiler_params=pltpu.CompilerParams(
            dimension_semantics=("arbitrary",),
        ),
    )(gathered, gathered, edge_attr, w1a, w1b, w1c, b1, w2, b2)


_TN = 400
_GN = _N // _TN


def _node_body(x, summed, cnt, w1a, w1b, b1, w2, b2, out):
    aggr = summed[...] / jnp.maximum(cnt[...], 1.0)
    h = jnp.dot(x[...], w1a[...], preferred_element_type=jnp.float32)
    h = h + jnp.dot(aggr, w1b[...], preferred_element_type=jnp.float32)
    h = h + b1[...]
    h = jnp.where(h > 0.0, h, jnp.exp(jnp.minimum(h, 0.0)) - 1.0)
    out[...] = jnp.dot(h, w2[...], preferred_element_type=jnp.float32) + b2[...]


def _node_mlp(x, summed, cnt, w1a, w1b, b1, w2, b2):
    return pl.pallas_call(
        _node_body,
        grid=(_GN,),
        in_specs=[
            pl.BlockSpec((_TN, _D), lambda i: (i, 0)),
            pl.BlockSpec((_TN, _D), lambda i: (i, 0)),
            pl.BlockSpec((_TN, 1), lambda i: (i, 0)),
            pl.BlockSpec((_D, _HID), lambda i: (0, 0)),
            pl.BlockSpec((_D, _HID), lambda i: (0, 0)),
            pl.BlockSpec((1, _HID), lambda i: (0, 0)),
            pl.BlockSpec((_HID, _D), lambda i: (0, 0)),
            pl.BlockSpec((1, _D), lambda i: (0, 0)),
        ],
        out_specs=pl.BlockSpec((_TN, _D), lambda i: (i, 0)),
        out_shape=jax.ShapeDtypeStruct((_N, _D), jnp.float32),
        compiler_params=pltpu.CompilerParams(
            dimension_semantics=("arbitrary",),
        ),
    )(x, summed, cnt, w1a, w1b, b1, w2, b2)


def kernel(x, edge_index, edge_attr, phi_w1, phi_b1, phi_w2, phi_b2,
           g_w1, g_b1, g_w2, g_b2):
    dst = edge_index[1]
    idx = jnp.concatenate([dst, edge_index[0]])
    gathered = _sc_gather(x, idx)

    msg = _edge_mlp(
        gathered, edge_attr,
        phi_w1[:_D].astype(jnp.bfloat16), phi_w1[_D:2 * _D].astype(jnp.bfloat16),
        phi_w1[2 * _D:],
        phi_b1.reshape(1, _HID), phi_w2, phi_b2.reshape(1, _D),
    )

    summed, cnt = _sc_scatter(msg, dst)

    return _node_mlp(
        x, summed, cnt.reshape(-1, 1),
        g_w1[:_D], g_w1[_D:], g_b1.reshape(1, _HID),
        g_w2, g_b2.reshape(1, _D),
    )


# speedup vs baseline: 1.0936x; 1.0936x over previous
import functools

import jax
import jax.numpy as jnp
from jax import lax
from jax.experimental import pallas as pl
from jax.experimental.pallas import tpu as pltpu
from jax.experimental.pallas import tpu_sc as plsc

_N = 10000
_E = 160000
_D = 256
_DE = 16
_HID = 512

_NC = 2
_NS = 16
_NW = _NC * _NS

_sc_params = pltpu.CompilerParams(needs_layout_passes=False)
_sc_mesh = plsc.VectorSubcoreMesh(core_axis_name="c", subcore_axis_name="s")

_GB = 2 * _E
_GPW = _GB // _NW
_GK = 80
_GCH = _GPW // _GK


@functools.partial(
    pl.kernel,
    mesh=_sc_mesh,
    compiler_params=_sc_params,
    out_type=jax.ShapeDtypeStruct((_GB, _D), jnp.float32),
    scratch_types=[
        pltpu.VMEM((_GPW,), jnp.int32),
        pltpu.VMEM((_GK, _D), jnp.float32),
        pltpu.VMEM((_GK, _D), jnp.float32),
        pltpu.SemaphoreType.DMA,
        pltpu.SemaphoreType.DMA,
        pltpu.SemaphoreType.DMA,
        pltpu.SemaphoreType.DMA,
    ],
)
def _sc_gather(x_hbm, idx_hbm, out_hbm, idx_v, rowsA, rowsB,
               gsemA, gsemB, wsemA, wsemB):
    c = lax.axis_index("c")
    s = lax.axis_index("s")
    wid = s * _NC + c
    base = wid * _GPW
    pltpu.sync_copy(idx_hbm.at[pl.ds(base, _GPW)], idx_v)
    pltpu.async_copy(x_hbm.at[idx_v.at[pl.ds(0, _GK)]], rowsA, gsemA)

    def super_step(k2, carry):
        for b in range(2):
            chunk = k2 * 2 + b
            rowsP, gsemP, wsemP = (
                (rowsA, gsemA, wsemA) if b == 0 else (rowsB, gsemB, wsemB)
            )
            rowsQ, gsemQ, wsemQ = (
                (rowsB, gsemB, wsemB) if b == 0 else (rowsA, gsemA, wsemA)
            )
            off = chunk * _GK

            @pl.when(chunk < _GCH)
            def _():
                pltpu.make_async_copy(
                    x_hbm.at[idx_v.at[pl.ds(off, _GK)]], rowsP, gsemP
                ).wait()

                @pl.when(chunk >= 1)
                def _():
                    pltpu.make_async_copy(
                        rowsQ, out_hbm.at[pl.ds(base + off - _GK, _GK)], wsemQ
                    ).wait()

                @pl.when(chunk + 1 < _GCH)
                def _():
                    pltpu.async_copy(
                        x_hbm.at[idx_v.at[pl.ds(off + _GK, _GK)]],
                        rowsQ, gsemQ,
                    )

                pltpu.async_copy(
                    rowsP, out_hbm.at[pl.ds(base + off, _GK)], wsemP
                )

            @pl.when(chunk == _GCH)
            def _():
                pltpu.make_async_copy(
                    rowsQ, out_hbm.at[pl.ds(base + off - _GK, _GK)], wsemQ
                ).wait()

        return carry

    lax.fori_loop(0, _GCH // 2 + 1, super_step, 0)


_OWN = 320
_SCC = 1600
_NCH = _E // _SCC
_SK = 48
_EC = _SCC + _SK
_AR = 336


@functools.partial(
    pl.kernel,
    mesh=_sc_mesh,
    compiler_params=_sc_params,
    out_type=(
        jax.ShapeDtypeStruct((_NW * _OWN, _D), jnp.float32),
        jax.ShapeDtypeStruct((_NW * _OWN,), jnp.float32),
    ),
    scratch_types=[
        pltpu.VMEM((_SCC,), jnp.int32),
        pltpu.VMEM((_SCC,), jnp.int32),
        pltpu.VMEM((_EC,), jnp.int32),
        pltpu.VMEM((_EC,), jnp.int32),
        pltpu.VMEM((_EC,), jnp.int32),
        pltpu.VMEM((_EC,), jnp.int32),
        pltpu.VMEM((_SK, _D), jnp.float32),
        pltpu.VMEM((_SK, _D), jnp.float32),
        pltpu.VMEM((_AR, _D), jnp.float32),
        pltpu.VMEM((16 * _OWN,), jnp.float32),
        pltpu.VMEM((_OWN,), jnp.float32),
        pltpu.SemaphoreType.DMA,
        pltpu.SemaphoreType.DMA,
        pltpu.SemaphoreType.DMA,
        pltpu.SemaphoreType.DMA,
    ],
)
def _sc_scatter(msg_hbm, dst_hbm, acc_out, cnt_out,
                dstA, dstB, elA, elB, llA, llB, mbA, mbB,
                acc, lcnt, cntf, dsemA, dsemB, msemA, msemB):
    c = lax.axis_index("c")
    s = lax.axis_index("s")
    wid = s * _NC + c
    nbase = wid * _OWN
    zero16 = jnp.zeros((16,), jnp.float32)
    ones16 = jnp.ones((16,), jnp.float32)
    iota16 = lax.iota(jnp.int32, 16)

    def zrow(r, carry):
        for l in range(_D // 16):
            acc[r, pl.ds(l * 16, 16)] = zero16
        return carry

    lax.fori_loop(0, _AR, zrow, 0)

    def zcnt(r, carry):
        lcnt[pl.ds(r * 16, 16)] = zero16
        return carry

    lax.fori_loop(0, 16 * _OWN // 16, zcnt, 0)

    def compact(chunk, dstb, elb, llb, msem, mbb):
        ebase = chunk * _SCC

        def grp(g, carry):
            pos, evec = carry
            for u in range(2):
                dv = dstb[pl.ds(g * 32 + u * 16, 16)]
                ok = (dv >= nbase) & (dv < nbase + _OWN)
                lid = jnp.where(ok, dv - nbase, 0)
                plsc.addupdate_scatter(lcnt, [iota16 * _OWN + lid], ones16,
                                       mask=ok)
                rank = plsc.cumsum(
                    jnp.where(ok, jnp.int32(1), jnp.int32(0))) - 1
                slot = jnp.maximum(pos + rank, 0)
                plsc.store_scatter(elb, [slot], evec, mask=ok)
                pc = plsc.all_reduce_population_count(ok)
                pos = pos + pc
                evec = evec + 16
            return pos, evec

        pos, _ = lax.fori_loop(0, _SCC // 32, grp,
                               (jnp.zeros((16,), jnp.int32), ebase + iota16))
        m = jnp.max(pos)
        padv = jnp.full((16,), ebase, jnp.int32)
        for j in range(_SK // 16):
            plsc.store_scatter(elb, [m + j * 16 + iota16], padv)

        def lids(r, carry):
            eidv = elb[pl.ds(r * 16, 16)]
            lid16 = plsc.load_gather(dstb, [eidv - ebase]) - nbase
            llb[pl.ds(r * 16, 16)] = lid16
            return carry

        lax.fori_loop(0, (m + 15) // 16, lids, 0)
        dumv = jnp.full((16,), _OWN, jnp.int32)
        for j in range(_SK // 16):
            plsc.store_scatter(llb, [m + j * 16 + iota16], dumv)
        cp = pltpu.async_copy(msg_hbm.at[elb.at[pl.ds(0, _SK)]], mbb, msem)
        del cp
        return m

    def accumulate(m, elb, llb, msem, mbb):
        pltpu.make_async_copy(msg_hbm.at[elb.at[pl.ds(0, _SK)]], mbb,
                              msem).wait()

        def dorows(t, carry):
            nrow = jnp.minimum(_SK, m - t * _SK)

            def row(j, carry2):
                for u in range(2):
                    jj = j * 2 + u
                    rid = plsc.load_gather(
                        llb, [jnp.full((16,), t * _SK + jj, jnp.int32)]
                    )
                    for l in range(_D // 16):
                        plsc.addupdate_scatter(
                            acc, [rid, iota16 + l * 16],
                            mbb[jj, pl.ds(l * 16, 16)]
                        )
                return carry2

            lax.fori_loop(0, (nrow + 1) // 2, row, 0)
            return carry

        dorows(jnp.int32(0), 0)

        def extra(t, carry):
            pltpu.async_copy(
                msg_hbm.at[elb.at[pl.ds(t * _SK, _SK)]], mbb, msem
            ).wait()
            dorows(t, carry)
            return carry

        lax.fori_loop(1, (m + _SK - 1) // _SK, extra, 0)

    pltpu.async_copy(dst_hbm.at[pl.ds(0, _SCC)], dstA, dsemA)

    def super_step(k2, m_prev):
        for b in range(2):
            chunk = k2 * 2 + b
            dstb, elb, llb, msem, mbb = (
                (dstA, elA, llA, msemA, mbA) if b == 0
                else (dstB, elB, llB, msemB, mbB)
            )
            dsem = dsemA if b == 0 else dsemB
            dsemN = dsemB if b == 0 else dsemA
            dstN = dstB if b == 0 else dstA
            elQ, llQ, msemQ, mbQ = (
                (elB, llB, msemB, mbB) if b == 0
                else (elA, llA, msemA, mbA)
            )

            @pl.when(chunk < _NCH)
            def _():
                pltpu.make_async_copy(
                    dst_hbm.at[pl.ds(chunk * _SCC, _SCC)], dstb, dsem
                ).wait()

                @pl.when(chunk + 1 < _NCH)
                def _():
                    pltpu.async_copy(
                        dst_hbm.at[pl.ds((chunk + 1) * _SCC, _SCC)],
                        dstN, dsemN,
                    )

            m_new = lax.cond(
                chunk < _NCH,
                lambda: compact(chunk, dstb, elb, llb, msem, mbb),
                lambda: jnp.int32(0),
            )

            @pl.when((chunk >= 1) & (chunk <= _NCH))
            def _():
                accumulate(m_prev, elQ, llQ, msemQ, mbQ)

            m_prev = m_new
        return m_prev

    lax.fori_loop(0, _NCH // 2 + 1, super_step, jnp.int32(0))

    def cred(g, carry):
        tot = zero16
        for lane in range(16):
            tot = tot + lcnt[pl.ds(lane * _OWN + g * 16, 16)]
        cntf[pl.ds(g * 16, 16)] = tot
        return carry

    lax.fori_loop(0, _OWN // 16, cred, 0)
    pltpu.sync_copy(acc.at[pl.ds(0, _OWN)], acc_out.at[pl.ds(nbase, _OWN)])
    pltpu.sync_copy(cntf, cnt_out.at[pl.ds(nbase, _OWN)])


_TE = 640
_GE = _E // _TE


def _edge_body(xi, xj, ea, w1a, w1b, w1c, b1, w2, b2, out):
    bf = jnp.bfloat16
    h = jnp.dot(xi[...].astype(bf), w1a[...], preferred_element_type=jnp.float32)
    h = h + jnp.dot(xj[...].astype(bf), w1b[...], preferred_element_type=jnp.float32)
    h = h + jnp.dot(ea[...], w1c[...], preferred_element_type=jnp.float32)
    h = h + b1[...]
    h = jnp.where(h > 0.0, h, jnp.exp(jnp.minimum(h, 0.0)) - 1.0)
    out[...] = jnp.dot(h.astype(bf), w2[...].astype(bf),
                       preferred_element_type=jnp.float32) + b2[...]


def _edge_mlp(gathered, edge_attr, w1a, w1b, w1c, b1, w2, b2):
    return pl.pallas_call(
        _edge_body,
        grid=(_GE,),
        in_specs=[
            pl.BlockSpec((_TE, _D), lambda i: (i, 0)),
            pl.BlockSpec((_TE, _D), lambda i: (i + _GE, 0)),
            pl.BlockSpec((_TE, _DE), lambda i: (i, 0)),
            pl.BlockSpec((_D, _HID), lambda i: (0, 0)),
            pl.BlockSpec((_D, _HID), lambda i: (0, 0)),
            pl.BlockSpec((_DE, _HID), lambda i: (0, 0)),
            pl.BlockSpec((1, _HID), lambda i: (0, 0)),
            pl.BlockSpec((_HID, _D), lambda i: (0, 0)),
            pl.BlockSpec((1, _D), lambda i: (0, 0)),
        ],
        out_specs=pl.BlockSpec((_TE, _D), lambda i: (i, 0)),
        out_shape=jax.ShapeDtypeStruct((_E, _D), jnp.float32),
        compiler_params=pltpu.CompilerParams(
            dimension_semantics=("arbitrary",),
        ),
    )(gathered, gathered, edge_attr, w1a, w1b, w1c, b1, w2, b2)


_TN = 400
_GN = _N // _TN


def _node_body(x, summed, cnt, w1a, w1b, b1, w2, b2, out):
    aggr = summed[...] / jnp.maximum(cnt[...], 1.0)
    h = jnp.dot(x[...], w1a[...], preferred_element_type=jnp.float32)
    h = h + jnp.dot(aggr, w1b[...], preferred_element_type=jnp.float32)
    h = h + b1[...]
    h = jnp.where(h > 0.0, h, jnp.exp(jnp.minimum(h, 0.0)) - 1.0)
    out[...] = jnp.dot(h, w2[...], preferred_element_type=jnp.float32) + b2[...]


def _node_mlp(x, summed, cnt, w1a, w1b, b1, w2, b2):
    return pl.pallas_call(
        _node_body,
        grid=(_GN,),
        in_specs=[
            pl.BlockSpec((_TN, _D), lambda i: (i, 0)),
            pl.BlockSpec((_TN, _D), lambda i: (i, 0)),
            pl.BlockSpec((_TN, 1), lambda i: (i, 0)),
            pl.BlockSpec((_D, _HID), lambda i: (0, 0)),
            pl.BlockSpec((_D, _HID), lambda i: (0, 0)),
            pl.BlockSpec((1, _HID), lambda i: (0, 0)),
            pl.BlockSpec((_HID, _D), lambda i: (0, 0)),
            pl.BlockSpec((1, _D), lambda i: (0, 0)),
        ],
        out_specs=pl.BlockSpec((_TN, _D), lambda i: (i, 0)),
        out_shape=jax.ShapeDtypeStruct((_N, _D), jnp.float32),
        compiler_params=pltpu.CompilerParams(
            dimension_semantics=("arbitrary",),
        ),
    )(x, summed, cnt, w1a, w1b, b1, w2, b2)


def kernel(x, edge_index, edge_attr, phi_w1, phi_b1, phi_w2, phi_b2,
           g_w1, g_b1, g_w2, g_b2):
    dst = edge_index[1]
    idx = jnp.concatenate([dst, edge_index[0]])
    gathered = _sc_gather(x, idx)

    msg = _edge_mlp(
        gathered, edge_attr,
        phi_w1[:_D].astype(jnp.bfloat16), phi_w1[_D:2 * _D].astype(jnp.bfloat16),
        phi_w1[2 * _D:],
        phi_b1.reshape(1, _HID), phi_w2, phi_b2.reshape(1, _D),
    )

    summed, cnt = _sc_scatter(msg, dst)

    return _node_mlp(
        x, summed, cnt.reshape(-1, 1),
        g_w1[:_D], g_w1[_D:], g_b1.reshape(1, _HID),
        g_w2, g_b2.reshape(1, _D),
    )


# speedup vs baseline: 1.1307x; 1.0339x over previous
import functools

import jax
import jax.numpy as jnp
from jax import lax
from jax.experimental import pallas as pl
from jax.experimental.pallas import tpu as pltpu
from jax.experimental.pallas import tpu_sc as plsc

_N = 10000
_E = 160000
_D = 256
_DE = 16
_HID = 512

_NC = 2
_NS = 16
_NW = _NC * _NS

_sc_params = pltpu.CompilerParams(needs_layout_passes=False)
_sc_mesh = plsc.VectorSubcoreMesh(core_axis_name="c", subcore_axis_name="s")

_EH = _E // 2
_GB = 2 * _EH
_GPW = _GB // _NW
_GK = 40
_GCH = _GPW // _GK


@functools.partial(
    pl.kernel,
    mesh=_sc_mesh,
    compiler_params=_sc_params,
    out_type=jax.ShapeDtypeStruct((_GB, _D), jnp.float32),
    scratch_types=[
        pltpu.VMEM((_GPW,), jnp.int32),
        pltpu.VMEM((_GK, _D), jnp.float32),
        pltpu.VMEM((_GK, _D), jnp.float32),
        pltpu.SemaphoreType.DMA,
        pltpu.SemaphoreType.DMA,
        pltpu.SemaphoreType.DMA,
        pltpu.SemaphoreType.DMA,
    ],
)
def _sc_gather(x_hbm, idx_hbm, out_hbm, idx_v, rowsA, rowsB,
               gsemA, gsemB, wsemA, wsemB):
    c = lax.axis_index("c")
    s = lax.axis_index("s")
    wid = s * _NC + c
    base = wid * _GPW
    pltpu.sync_copy(idx_hbm.at[pl.ds(base, _GPW)], idx_v)
    pltpu.async_copy(x_hbm.at[idx_v.at[pl.ds(0, _GK)]], rowsA, gsemA)

    def super_step(k2, carry):
        for b in range(2):
            chunk = k2 * 2 + b
            rowsP, gsemP, wsemP = (
                (rowsA, gsemA, wsemA) if b == 0 else (rowsB, gsemB, wsemB)
            )
            rowsQ, gsemQ, wsemQ = (
                (rowsB, gsemB, wsemB) if b == 0 else (rowsA, gsemA, wsemA)
            )
            off = chunk * _GK

            @pl.when(chunk < _GCH)
            def _():
                pltpu.make_async_copy(
                    x_hbm.at[idx_v.at[pl.ds(off, _GK)]], rowsP, gsemP
                ).wait()

                @pl.when(chunk >= 1)
                def _():
                    pltpu.make_async_copy(
                        rowsQ, out_hbm.at[pl.ds(base + off - _GK, _GK)], wsemQ
                    ).wait()

                @pl.when(chunk + 1 < _GCH)
                def _():
                    pltpu.async_copy(
                        x_hbm.at[idx_v.at[pl.ds(off + _GK, _GK)]],
                        rowsQ, gsemQ,
                    )

                pltpu.async_copy(
                    rowsP, out_hbm.at[pl.ds(base + off, _GK)], wsemP
                )

            @pl.when(chunk == _GCH)
            def _():
                pltpu.make_async_copy(
                    rowsQ, out_hbm.at[pl.ds(base + off - _GK, _GK)], wsemQ
                ).wait()

        return carry

    lax.fori_loop(0, _GCH // 2 + 1, super_step, 0)


_OWN = 320
_SCC = 1600
_NCH = _EH // _SCC
_SK = 48
_EC = _SCC + _SK
_AR = 336


@functools.partial(
    pl.kernel,
    mesh=_sc_mesh,
    compiler_params=_sc_params,
    out_type=(
        jax.ShapeDtypeStruct((_NW * _OWN, _D), jnp.float32),
        jax.ShapeDtypeStruct((_NW * _OWN,), jnp.float32),
    ),
    scratch_types=[
        pltpu.VMEM((_SCC,), jnp.int32),
        pltpu.VMEM((_SCC,), jnp.int32),
        pltpu.VMEM((_EC,), jnp.int32),
        pltpu.VMEM((_EC,), jnp.int32),
        pltpu.VMEM((_EC,), jnp.int32),
        pltpu.VMEM((_EC,), jnp.int32),
        pltpu.VMEM((_SK, _D), jnp.float32),
        pltpu.VMEM((_SK, _D), jnp.float32),
        pltpu.VMEM((_AR, _D), jnp.float32),
        pltpu.VMEM((16 * _OWN,), jnp.float32),
        pltpu.VMEM((_OWN,), jnp.float32),
        pltpu.SemaphoreType.DMA,
        pltpu.SemaphoreType.DMA,
        pltpu.SemaphoreType.DMA,
        pltpu.SemaphoreType.DMA,
    ],
)
def _sc_scatter(msg_hbm, dst_hbm, acc_out, cnt_out,
                dstA, dstB, elA, elB, llA, llB, mbA, mbB,
                acc, lcnt, cntf, dsemA, dsemB, msemA, msemB):
    c = lax.axis_index("c")
    s = lax.axis_index("s")
    wid = s * _NC + c
    nbase = wid * _OWN
    zero16 = jnp.zeros((16,), jnp.float32)
    ones16 = jnp.ones((16,), jnp.float32)
    iota16 = lax.iota(jnp.int32, 16)

    def zrow(r, carry):
        for l in range(_D // 16):
            acc[r, pl.ds(l * 16, 16)] = zero16
        return carry

    lax.fori_loop(0, _AR, zrow, 0)

    def zcnt(r, carry):
        lcnt[pl.ds(r * 16, 16)] = zero16
        return carry

    lax.fori_loop(0, 16 * _OWN // 16, zcnt, 0)

    def compact(chunk, dstb, elb, llb, msem, mbb):
        ebase = chunk * _SCC

        def grp(g, carry):
            pos, evec = carry
            for u in range(2):
                dv = dstb[pl.ds(g * 32 + u * 16, 16)]
                ok = (dv >= nbase) & (dv < nbase + _OWN)
                lid = jnp.where(ok, dv - nbase, 0)
                plsc.addupdate_scatter(lcnt, [iota16 * _OWN + lid], ones16,
                                       mask=ok)
                rank = plsc.cumsum(
                    jnp.where(ok, jnp.int32(1), jnp.int32(0))) - 1
                slot = jnp.maximum(pos + rank, 0)
                plsc.store_scatter(elb, [slot], evec, mask=ok)
                pc = plsc.all_reduce_population_count(ok)
                pos = pos + pc
                evec = evec + 16
            return pos, evec

        pos, _ = lax.fori_loop(0, _SCC // 32, grp,
                               (jnp.zeros((16,), jnp.int32), ebase + iota16))
        m = jnp.max(pos)
        padv = jnp.full((16,), ebase, jnp.int32)
        for j in range(_SK // 16):
            plsc.store_scatter(elb, [m + j * 16 + iota16], padv)

        def lids(r, carry):
            eidv = elb[pl.ds(r * 16, 16)]
            lid16 = plsc.load_gather(dstb, [eidv - ebase]) - nbase
            llb[pl.ds(r * 16, 16)] = lid16
            return carry

        lax.fori_loop(0, (m + 15) // 16, lids, 0)
        dumv = jnp.full((16,), _OWN, jnp.int32)
        for j in range(_SK // 16):
            plsc.store_scatter(llb, [m + j * 16 + iota16], dumv)
        cp = pltpu.async_copy(msg_hbm.at[elb.at[pl.ds(0, _SK)]], mbb, msem)
        del cp
        return m

    def accumulate(m, elb, llb, msem, mbb):
        pltpu.make_async_copy(msg_hbm.at[elb.at[pl.ds(0, _SK)]], mbb,
                              msem).wait()

        def dorows(t, carry):
            nrow = jnp.minimum(_SK, m - t * _SK)

            def row(j, carry2):
                for u in range(2):
                    jj = j * 2 + u
                    rid = plsc.load_gather(
                        llb, [jnp.full((16,), t * _SK + jj, jnp.int32)]
                    )
                    for l in range(_D // 16):
                        plsc.addupdate_scatter(
                            acc, [rid, iota16 + l * 16],
                            mbb[jj, pl.ds(l * 16, 16)]
                        )
                return carry2

            lax.fori_loop(0, (nrow + 1) // 2, row, 0)
            return carry

        dorows(jnp.int32(0), 0)

        def extra(t, carry):
            pltpu.async_copy(
                msg_hbm.at[elb.at[pl.ds(t * _SK, _SK)]], mbb, msem
            ).wait()
            dorows(t, carry)
            return carry

        lax.fori_loop(1, (m + _SK - 1) // _SK, extra, 0)

    pltpu.async_copy(dst_hbm.at[pl.ds(0, _SCC)], dstA, dsemA)

    def super_step(k2, m_prev):
        for b in range(2):
            chunk = k2 * 2 + b
            dstb, elb, llb, msem, mbb = (
                (dstA, elA, llA, msemA, mbA) if b == 0
                else (dstB, elB, llB, msemB, mbB)
            )
            dsem = dsemA if b == 0 else dsemB
            dsemN = dsemB if b == 0 else dsemA
            dstN = dstB if b == 0 else dstA
            elQ, llQ, msemQ, mbQ = (
                (elB, llB, msemB, mbB) if b == 0
                else (elA, llA, msemA, mbA)
            )

            @pl.when(chunk < _NCH)
            def _():
                pltpu.make_async_copy(
                    dst_hbm.at[pl.ds(chunk * _SCC, _SCC)], dstb, dsem
                ).wait()

                @pl.when(chunk + 1 < _NCH)
                def _():
                    pltpu.async_copy(
                        dst_hbm.at[pl.ds((chunk + 1) * _SCC, _SCC)],
                        dstN, dsemN,
                    )

            m_new = lax.cond(
                chunk < _NCH,
                lambda: compact(chunk, dstb, elb, llb, msem, mbb),
                lambda: jnp.int32(0),
            )

            @pl.when((chunk >= 1) & (chunk <= _NCH))
            def _():
                accumulate(m_prev, elQ, llQ, msemQ, mbQ)

            m_prev = m_new
        return m_prev

    lax.fori_loop(0, _NCH // 2 + 1, super_step, jnp.int32(0))

    def cred(g, carry):
        tot = zero16
        for lane in range(16):
            tot = tot + lcnt[pl.ds(lane * _OWN + g * 16, 16)]
        cntf[pl.ds(g * 16, 16)] = tot
        return carry

    lax.fori_loop(0, _OWN // 16, cred, 0)
    pltpu.sync_copy(acc.at[pl.ds(0, _OWN)], acc_out.at[pl.ds(nbase, _OWN)])
    pltpu.sync_copy(cntf, cnt_out.at[pl.ds(nbase, _OWN)])


_TE = 640
_GE = _EH // _TE


def _edge_body(xi, xj, ea, w1a, w1b, w1c, b1, w2, b2, out):
    bf = jnp.bfloat16
    h = jnp.dot(xi[...].astype(bf), w1a[...], preferred_element_type=jnp.float32)
    h = h + jnp.dot(xj[...].astype(bf), w1b[...], preferred_element_type=jnp.float32)
    h = h + jnp.dot(ea[...], w1c[...], preferred_element_type=jnp.float32)
    h = h + b1[...]
    h = jnp.where(h > 0.0, h, jnp.exp(jnp.minimum(h, 0.0)) - 1.0)
    out[...] = jnp.dot(h.astype(bf), w2[...].astype(bf),
                       preferred_element_type=jnp.float32) + b2[...]


def _edge_mlp(gathered, edge_attr, w1a, w1b, w1c, b1, w2, b2):
    return pl.pallas_call(
        _edge_body,
        grid=(_GE,),
        in_specs=[
            pl.BlockSpec((_TE, _D), lambda i: (i, 0)),
            pl.BlockSpec((_TE, _D), lambda i: (i + _GE, 0)),
            pl.BlockSpec((_TE, _DE), lambda i: (i, 0)),
            pl.BlockSpec((_D, _HID), lambda i: (0, 0)),
            pl.BlockSpec((_D, _HID), lambda i: (0, 0)),
            pl.BlockSpec((_DE, _HID), lambda i: (0, 0)),
            pl.BlockSpec((1, _HID), lambda i: (0, 0)),
            pl.BlockSpec((_HID, _D), lambda i: (0, 0)),
            pl.BlockSpec((1, _D), lambda i: (0, 0)),
        ],
        out_specs=pl.BlockSpec((_TE, _D), lambda i: (i, 0)),
        out_shape=jax.ShapeDtypeStruct((_EH, _D), jnp.float32),
        compiler_params=pltpu.CompilerParams(
            dimension_semantics=("arbitrary",),
        ),
    )(gathered, gathered, edge_attr, w1a, w1b, w1c, b1, w2, b2)


_TN = 400
_GN = _N // _TN


def _node_body(x, s1, s2, c1, c2, w1a, w1b, b1, w2, b2, out):
    aggr = (s1[...] + s2[...]) / jnp.maximum(c1[...] + c2[...], 1.0)
    h = jnp.dot(x[...], w1a[...], preferred_element_type=jnp.float32)
    h = h + jnp.dot(aggr, w1b[...], preferred_element_type=jnp.float32)
    h = h + b1[...]
    h = jnp.where(h > 0.0, h, jnp.exp(jnp.minimum(h, 0.0)) - 1.0)
    out[...] = jnp.dot(h, w2[...], preferred_element_type=jnp.float32) + b2[...]


def _node_mlp(x, s1, s2, c1, c2, w1a, w1b, b1, w2, b2):
    return pl.pallas_call(
        _node_body,
        grid=(_GN,),
        in_specs=[
            pl.BlockSpec((_TN, _D), lambda i: (i, 0)),
            pl.BlockSpec((_TN, _D), lambda i: (i, 0)),
            pl.BlockSpec((_TN, _D), lambda i: (i, 0)),
            pl.BlockSpec((_TN, 1), lambda i: (i, 0)),
            pl.BlockSpec((_TN, 1), lambda i: (i, 0)),
            pl.BlockSpec((_D, _HID), lambda i: (0, 0)),
            pl.BlockSpec((_D, _HID), lambda i: (0, 0)),
            pl.BlockSpec((1, _HID), lambda i: (0, 0)),
            pl.BlockSpec((_HID, _D), lambda i: (0, 0)),
            pl.BlockSpec((1, _D), lambda i: (0, 0)),
        ],
        out_specs=pl.BlockSpec((_TN, _D), lambda i: (i, 0)),
        out_shape=jax.ShapeDtypeStruct((_N, _D), jnp.float32),
        compiler_params=pltpu.CompilerParams(
            dimension_semantics=("arbitrary",),
        ),
    )(x, s1, s2, c1, c2, w1a, w1b, b1, w2, b2)


def kernel(x, edge_index, edge_attr, phi_w1, phi_b1, phi_w2, phi_b2,
           g_w1, g_b1, g_w2, g_b2):
    dst = edge_index[1]
    src_idx = edge_index[0]
    w1a = phi_w1[:_D].astype(jnp.bfloat16)
    w1b = phi_w1[_D:2 * _D].astype(jnp.bfloat16)
    w1c = phi_w1[2 * _D:]
    b1 = phi_b1.reshape(1, _HID)
    b2 = phi_b2.reshape(1, _D)

    halves = []
    for h in range(2):
        dh = lax.slice_in_dim(dst, h * _EH, (h + 1) * _EH)
        sh = lax.slice_in_dim(src_idx, h * _EH, (h + 1) * _EH)
        eah = lax.slice_in_dim(edge_attr, h * _EH, (h + 1) * _EH)
        gathered = _sc_gather(x, jnp.concatenate([dh, sh]))
        msg = _edge_mlp(gathered, eah, w1a, w1b, w1c, b1, phi_w2, b2)
        halves.append(_sc_scatter(msg, dh))

    (s1, c1), (s2, c2) = halves
    return _node_mlp(
        x, s1, s2, c1.reshape(-1, 1), c2.reshape(-1, 1),
        g_w1[:_D], g_w1[_D:], g_b1.reshape(1, _HID),
        g_w2, g_b2.reshape(1, _D),
    )
